# baseline (speedup 1.0000x reference)
import functools

import jax
import jax.numpy as jnp
from jax import lax
from jax.experimental import pallas as pl
from jax.experimental.pallas import tpu as pltpu
from jax.experimental.pallas import tpu_sc as plsc

_INFO = plsc.get_sparse_core_info()
_NC = _INFO.num_cores
_NS = _INFO.num_subcores
_NW = _NC * _NS
_L = _INFO.num_lanes

_E = 320000
_D = 128
_PER_W = _E // _NW
_C = 80
_NCHUNK = _PER_W // _C


def _make_sc_kernel():
    mesh = plsc.VectorSubcoreMesh(core_axis_name="c", subcore_axis_name="s")

    @functools.partial(
        pl.kernel,
        mesh=mesh,
        out_type=jax.ShapeDtypeStruct((_NW, _NCHUNK, _C), jnp.float32),
        compiler_params=pltpu.CompilerParams(needs_layout_passes=False, use_tc_tiling_on_sc=False),
        scratch_types=[
            pltpu.VMEM((_NCHUNK, _C), jnp.int32),
            pltpu.VMEM((_NCHUNK, _C), jnp.int32),
            pltpu.VMEM((_C, _D // 2), jnp.int32),
            pltpu.VMEM((_C, _D // 2), jnp.int32),
            pltpu.VMEM((_C, _D // 2), jnp.int32),
            pltpu.VMEM((_C, _D // 2), jnp.int32),
            pltpu.VMEM((_NCHUNK, _C), jnp.float32),
            pltpu.VMEM_SHARED((10000, _D // 2), jnp.int32),
            pltpu.SemaphoreType.DMA,
            pltpu.SemaphoreType.DMA,
            pltpu.SemaphoreType.DMA,
            pltpu.SemaphoreType.DMA,
        ],
    )
    def k(h_hbm, u_hbm, v_hbm, out_hbm,
          iu, iv, ru0, rv0, ru1, rv1, scores, sh, su0, sv0, su1, sv1):
        wid = lax.axis_index("s") * _NC + lax.axis_index("c")
        lanes = lax.iota(jnp.int32, _L)

        @pl.when(lax.axis_index("s") == 0)
        def _stage_h():
            pltpu.sync_copy(h_hbm, sh)

        pltpu.sync_copy(u_hbm.at[wid], iu)
        pltpu.sync_copy(v_hbm.at[wid], iv)
        plsc.subcore_barrier()

        def start(c, ru, rv, su, sv):
            pltpu.async_copy(sh.at[iu.at[c]], ru, su)
            pltpu.async_copy(sh.at[iv.at[c]], rv, sv)

        def wait(c, ru, rv, su, sv):
            pltpu.make_async_copy(sh.at[iu.at[c]], ru, su).wait()
            pltpu.make_async_copy(sh.at[iv.at[c]], rv, sv).wait()

        def compute(c, ru, rv):
            @plsc.parallel_loop(0, _C // _L, 1, unroll=5)
            def group_body(g):
                accs = []
                for j in range(_L):
                    e = g * _L + j
                    parts = []
                    for t in range(_D // (2 * _L)):
                        bu = plsc.bitcast(ru[e, pl.ds(t * _L, _L)], jnp.bfloat16)
                        bv = plsc.bitcast(rv[e, pl.ds(t * _L, _L)], jnp.bfloat16)
                        p0, p1 = plsc.unpack(bu * bv, format=plsc.PackFormat.INTERLEAVED)
                        parts.append(p0 + p1)
                    accs.append((parts[0] + parts[1]) + (parts[2] + parts[3]))
                for bit in (1, 2, 4, 8):
                    sel = (lanes & bit) == 0
                    perm = jnp.bitwise_xor(lanes, bit)
                    nxt = []
                    for m in range(0, len(accs), 2):
                        a = accs[m] + jnp.take_along_axis(accs[m], perm, axis=0)
                        b = accs[m + 1] + jnp.take_along_axis(accs[m + 1], perm, axis=0)
                        nxt.append(jnp.where(sel, a, b))
                    accs = nxt
                scores[c, pl.ds(g * _L, _L)] = accs[0]

        start(0, ru0, rv0, su0, sv0)

        def body(c2, _):
            ca = 2 * c2
            cb = ca + 1
            start(cb, ru1, rv1, su1, sv1)
            wait(ca, ru0, rv0, su0, sv0)
            compute(ca, ru0, rv0)
            start(ca + 2, ru0, rv0, su0, sv0)
            wait(cb, ru1, rv1, su1, sv1)
            compute(cb, ru1, rv1)
            return 0

        lax.fori_loop(0, (_NCHUNK - 1) // 2, body, 0)
        wait(_NCHUNK - 1, ru0, rv0, su0, sv0)
        compute(_NCHUNK - 1, ru0, rv0)

        pltpu.sync_copy(scores, out_hbm.at[wid])

    return k


_sc_kernel = _make_sc_kernel()


@jax.jit
def kernel(h, edge_index):
    ei = edge_index.astype(jnp.int32).reshape(2, _NW, _NCHUNK, _C)
    hb = h.astype(jnp.bfloat16).reshape(h.shape[0], _D // 2, 2)
    h_packed = lax.bitcast_convert_type(hb, jnp.int32)
    out = _sc_kernel(h_packed, ei[0], ei[1])
    return out.reshape(_E)


# speedup vs baseline: 1.0180x; 1.0180x over previous
import functools

import jax
import jax.numpy as jnp
from jax import lax
from jax.experimental import pallas as pl
from jax.experimental.pallas import tpu as pltpu
from jax.experimental.pallas import tpu_sc as plsc

_INFO = plsc.get_sparse_core_info()
_NC = _INFO.num_cores
_NS = _INFO.num_subcores
_NW = _NC * _NS
_L = _INFO.num_lanes

_E = 320000
_D = 128
_PER_W = _E // _NW
_C = 80
_NCHUNK = _PER_W // _C


def _make_sc_kernel():
    mesh = plsc.VectorSubcoreMesh(core_axis_name="c", subcore_axis_name="s")

    @functools.partial(
        pl.kernel,
        mesh=mesh,
        out_type=jax.ShapeDtypeStruct((_NW, _NCHUNK, _C), jnp.float32),
        compiler_params=pltpu.CompilerParams(needs_layout_passes=False, use_tc_tiling_on_sc=False),
        scratch_types=[
            pltpu.VMEM((_NCHUNK, _C), jnp.int32),
            pltpu.VMEM((_NCHUNK, _C), jnp.int32),
            pltpu.VMEM((_C, _D // 2), jnp.int32),
            pltpu.VMEM((_C, _D // 2), jnp.int32),
            pltpu.VMEM((_C, _D // 2), jnp.int32),
            pltpu.VMEM((_C, _D // 2), jnp.int32),
            pltpu.VMEM((_NCHUNK, _C), jnp.float32),
            pltpu.VMEM_SHARED((10000, _D // 2), jnp.int32),
            pltpu.SemaphoreType.DMA,
            pltpu.SemaphoreType.DMA,
            pltpu.SemaphoreType.DMA,
            pltpu.SemaphoreType.DMA,
        ],
    )
    def k(h_hbm, u_hbm, v_hbm, out_hbm,
          iu, iv, ru0, rv0, ru1, rv1, scores, sh, su0, sv0, su1, sv1):
        wid = lax.axis_index("s") * _NC + lax.axis_index("c")
        lanes = lax.iota(jnp.int32, _L)

        @pl.when(lax.axis_index("s") == 0)
        def _stage_h():
            pltpu.sync_copy(h_hbm, sh)

        pltpu.sync_copy(u_hbm.at[wid], iu)
        pltpu.sync_copy(v_hbm.at[wid], iv)
        plsc.subcore_barrier()

        def start(c, ru, rv, su, sv):
            pltpu.async_copy(sh.at[iu.at[c]], ru, su)
            pltpu.async_copy(sh.at[iv.at[c]], rv, sv)

        def wait(c, ru, rv, su, sv):
            pltpu.make_async_copy(sh.at[iu.at[c]], ru, su).wait()
            pltpu.make_async_copy(sh.at[iv.at[c]], rv, sv).wait()

        def compute(c, ru, rv):
            @plsc.parallel_loop(0, _C // _L, 1)
            def group_body(g):
                accs = []
                for j in range(_L):
                    e = g * _L + j
                    parts = []
                    for t in range(_D // (2 * _L)):
                        bu = plsc.bitcast(ru[e, pl.ds(t * _L, _L)], jnp.bfloat16)
                        bv = plsc.bitcast(rv[e, pl.ds(t * _L, _L)], jnp.bfloat16)
                        p0, p1 = plsc.unpack(bu * bv, format=plsc.PackFormat.INTERLEAVED)
                        parts.append(p0 + p1)
                    accs.append((parts[0] + parts[1]) + (parts[2] + parts[3]))
                for bit in (1, 2, 4, 8):
                    sel = (lanes & bit) == 0
                    perm = jnp.bitwise_xor(lanes, bit)
                    nxt = []
                    for m in range(0, len(accs), 2):
                        a = accs[m] + jnp.take_along_axis(accs[m], perm, axis=0)
                        b = accs[m + 1] + jnp.take_along_axis(accs[m + 1], perm, axis=0)
                        nxt.append(jnp.where(sel, a, b))
                    accs = nxt
                scores[c, pl.ds(g * _L, _L)] = accs[0]

        start(0, ru0, rv0, su0, sv0)

        def body(c2, _):
            ca = 2 * c2
            cb = ca + 1
            start(cb, ru1, rv1, su1, sv1)
            wait(ca, ru0, rv0, su0, sv0)
            compute(ca, ru0, rv0)
            start(ca + 2, ru0, rv0, su0, sv0)
            wait(cb, ru1, rv1, su1, sv1)
            compute(cb, ru1, rv1)
            return 0

        lax.fori_loop(0, (_NCHUNK - 1) // 2, body, 0)
        wait(_NCHUNK - 1, ru0, rv0, su0, sv0)
        compute(_NCHUNK - 1, ru0, rv0)

        pltpu.sync_copy(scores, out_hbm.at[wid])

    return k


_sc_kernel = _make_sc_kernel()


@jax.jit
def kernel(h, edge_index):
    ei = edge_index.astype(jnp.int32).reshape(2, _NW, _NCHUNK, _C)
    hb = h.astype(jnp.bfloat16).reshape(h.shape[0], _D // 2, 2)
    h_packed = lax.bitcast_convert_type(hb, jnp.int32)
    out = _sc_kernel(h_packed, ei[0], ei[1])
    return out.reshape(_E)
